# baseline (speedup 1.0000x reference)
import functools

import jax
import jax.numpy as jnp
from jax import lax
from jax.experimental import pallas as pl
from jax.experimental.pallas import tpu as pltpu
from jax.experimental.pallas import tpu_sc as plsc

_THRESH = 0.7
_MIN_KEPT = 10000
_ALPHA = 0.25

_NC = 1
_NS = 16
_NW = _NC * _NS
_LANES = 16
_HW = 512 * 512
_TOTAL = 4 * _HW
_SC_ROWS = 64
_SC_N = 4 * _SC_ROWS * 512
_CHUNK = _SC_N // _NW
_UNROLL = 4
_STEPS = _CHUNK // (_LANES * _UNROLL)
_TC_ROWS_PB = 512 - _SC_ROWS
_TC_BLOCK_ROWS = 64
_TC_GRID_R = _TC_ROWS_PB // _TC_BLOCK_ROWS

_LOGIT_T = 0.8472978603872034


def _focal_and_kept(x, is_one):
    ns = jnp.where(is_one, -x, x)
    s = -ns
    kept = s <= _LOGIT_T
    u = jnp.exp(-jnp.abs(x))
    y = u / (2.0 + u)
    y2 = y * y
    log1pu = y * (2.0 + y2 * (2.0 / 3 + y2 * (2.0 / 5 + y2 * (
        2.0 / 7 + y2 * (2.0 / 9 + y2 * (2.0 / 11))))))
    bce = log1pu + jnp.maximum(ns, 0.0)
    inv = 1.0 / (1.0 + u)
    om = jnp.where(s >= 0.0, u * inv, inv)
    fl = (_ALPHA * bce) * (om * om)
    return fl, kept


@functools.partial(
    pl.kernel,
    out_type=jax.ShapeDtypeStruct((_NW, 2, _LANES), jnp.float32),
    mesh=plsc.VectorSubcoreMesh(core_axis_name="c", subcore_axis_name="s",
                                num_cores=_NC),
    scratch_types=[
        pltpu.VMEM((_CHUNK,), jnp.float32),
        pltpu.VMEM((_CHUNK,), jnp.int32),
        pltpu.VMEM((2, _LANES), jnp.float32),
    ],
)
def _sc_partial_sums(x_hbm, t_hbm, out_hbm, x_v, t_v, acc_v):
    wid = lax.axis_index("s") * _NC + lax.axis_index("c")
    pltpu.sync_copy(x_hbm.at[wid], x_v)
    pltpu.sync_copy(t_hbm.at[wid], t_v)

    def body(i, carry):
        acc_fl, acc_cnt = carry
        base = i * (_LANES * _UNROLL)
        for k in range(_UNROLL):
            off = base + k * _LANES
            x = x_v[pl.ds(off, _LANES)]
            t = t_v[pl.ds(off, _LANES)]
            fl, kept = _focal_and_kept(x, t == 1)
            acc_fl = acc_fl + jnp.where(kept, fl, 0.0)
            acc_cnt = acc_cnt + jnp.where(kept, 1.0, 0.0)
        return acc_fl, acc_cnt

    zero = jnp.zeros((_LANES,), jnp.float32)
    acc_fl, acc_cnt = lax.fori_loop(0, _STEPS, body, (zero, zero))
    acc_v[0, :] = acc_fl
    acc_v[1, :] = acc_cnt
    pltpu.sync_copy(acc_v, out_hbm.at[wid])


def _tc_body(x_ref, t_ref, s1_ref, s2_ref):
    i = pl.program_id(0) * pl.num_programs(1) + pl.program_id(1)
    x = x_ref[0, 0, :, :]
    t = t_ref[0, 0, :, :]
    is_one = t == 1
    ns = jnp.where(is_one, -x, x)
    kept = -ns <= _LOGIT_T
    bce = jnp.maximum(x, 0.0) - jnp.where(is_one, x, 0.0) + jnp.log1p(
        jnp.exp(-jnp.abs(x)))
    pt = jnp.exp(-bce)
    om = 1.0 - pt
    fl = (_ALPHA * bce) * (om * om)
    m = kept.astype(jnp.float32)
    ps1 = jnp.sum(fl * m)
    ps2 = jnp.sum(m)

    @pl.when(i == 0)
    def _():
        s1_ref[0, 0] = ps1
        s2_ref[0, 0] = ps2

    @pl.when(i > 0)
    def _():
        s1_ref[0, 0] += ps1
        s2_ref[0, 0] += ps2


def _tc_partial_sums(inp, tgt32):
    idx = lambda b, r: (b, 0, (_SC_ROWS // _TC_BLOCK_ROWS) + r, 0)
    spec = pl.BlockSpec((1, 1, _TC_BLOCK_ROWS, 512), idx)
    return pl.pallas_call(
        _tc_body,
        grid=(4, _TC_GRID_R),
        in_specs=[spec, spec],
        out_specs=[
            pl.BlockSpec(memory_space=pltpu.SMEM),
            pl.BlockSpec(memory_space=pltpu.SMEM),
        ],
        out_shape=[
            jax.ShapeDtypeStruct((1, 1), jnp.float32),
            jax.ShapeDtypeStruct((1, 1), jnp.float32),
        ],
    )(inp, tgt32)


def _aug_loss(inp, tgt, s1, s2):
    del s1, s2
    x = inp[:, 0, :, :].reshape(-1)
    t = tgt[:, 0, :, :].astype(jnp.float32).reshape(-1)
    probs = jax.nn.sigmoid(x)
    kept = jnp.where(t == 1.0, probs <= _THRESH,
                     jnp.where(t == 0.0, probs >= 1.0 - _THRESH, False))
    hardest = jnp.argsort(jnp.abs(probs - 0.5))[:_MIN_KEPT]
    kept = kept.at[hardest].set(True)
    bce = jnp.maximum(x, 0.0) - x * t + jnp.log1p(jnp.exp(-jnp.abs(x)))
    pt = jnp.exp(-bce)
    fl = _ALPHA * (1.0 - pt) ** 2.0 * bce
    m = kept.astype(jnp.float32)
    return jnp.sum(fl * m) / jnp.sum(m)


def kernel(input, target):
    n_terms = jnp.float32(input.shape[1] - 1)
    tgt32 = target.astype(jnp.int32)
    x_sc = input[:, 0, :_SC_ROWS, :].reshape(_NW, _CHUNK)
    t_sc = tgt32[:, 0, :_SC_ROWS, :].reshape(_NW, _CHUNK)
    parts = _sc_partial_sums(x_sc, t_sc)
    tc_s1, tc_s2 = _tc_partial_sums(input, tgt32)
    s2 = jnp.sum(parts[:, 1, :]) + tc_s2[0, 0]

    def _fast(inp, tgt, pr, a1, b2):
        del inp, tgt
        return (jnp.sum(pr[:, 0, :]) + a1) / b2

    loss = lax.cond(s2 < _MIN_KEPT,
                    lambda inp, tgt, pr, a1, b2: _aug_loss(inp, tgt, a1, b2),
                    _fast,
                    input, target, parts, tc_s1[0, 0], s2)
    return n_terms * loss


# speedup vs baseline: 1.0672x; 1.0672x over previous
import functools

import jax
import jax.numpy as jnp
from jax import lax
from jax.experimental import pallas as pl
from jax.experimental.pallas import tpu as pltpu
from jax.experimental.pallas import tpu_sc as plsc

_THRESH = 0.7
_MIN_KEPT = 10000
_ALPHA = 0.25

_NC = 2
_NS = 16
_NW = _NC * _NS
_LANES = 16
_HW = 512 * 512
_TOTAL = 4 * _HW
_SC_ROWS = 128
_SC_N = 4 * _SC_ROWS * 512
_CHUNK = _SC_N // _NW
_UNROLL = 4
_STEPS = _CHUNK // (_LANES * _UNROLL)
_TC_BLOCK_ROWS = 128
_TCA_ROW0 = _SC_ROWS
_TCA_BLOCKS = 2
_TCB_ROW0 = _SC_ROWS + _TCA_BLOCKS * _TC_BLOCK_ROWS
_TCB_BLOCKS = (512 - _TCB_ROW0) // _TC_BLOCK_ROWS

_LOGIT_T = 0.8472978603872034


def _focal_and_kept(x, is_one):
    ns = jnp.where(is_one, -x, x)
    s = -ns
    kept = s <= _LOGIT_T
    u = jnp.exp(-jnp.abs(x))
    y = u / (2.0 + u)
    y2 = y * y
    log1pu = y * (2.0 + y2 * (2.0 / 3 + y2 * (2.0 / 5 + y2 * (
        2.0 / 7 + y2 * (2.0 / 9 + y2 * (2.0 / 11))))))
    bce = log1pu + jnp.maximum(ns, 0.0)
    inv = 1.0 / (1.0 + u)
    om = jnp.where(s >= 0.0, u * inv, inv)
    fl = (_ALPHA * bce) * (om * om)
    return fl, kept


@functools.partial(
    pl.kernel,
    out_type=jax.ShapeDtypeStruct((_NW, 2, _LANES), jnp.float32),
    mesh=plsc.VectorSubcoreMesh(core_axis_name="c", subcore_axis_name="s",
                                num_cores=_NC),
    scratch_types=[
        pltpu.VMEM((_CHUNK,), jnp.float32),
        pltpu.VMEM((_CHUNK,), jnp.int32),
        pltpu.VMEM((2, _LANES), jnp.float32),
    ],
)
def _sc_partial_sums(x_hbm, t_hbm, out_hbm, x_v, t_v, acc_v):
    wid = lax.axis_index("s") * _NC + lax.axis_index("c")
    pltpu.sync_copy(x_hbm.at[wid], x_v)
    pltpu.sync_copy(t_hbm.at[wid], t_v)

    def body(i, carry):
        acc_fl, acc_cnt = carry
        base = i * (_LANES * _UNROLL)
        for k in range(_UNROLL):
            off = base + k * _LANES
            x = x_v[pl.ds(off, _LANES)]
            t = t_v[pl.ds(off, _LANES)]
            fl, kept = _focal_and_kept(x, t == 1)
            acc_fl = acc_fl + jnp.where(kept, fl, 0.0)
            acc_cnt = acc_cnt + jnp.where(kept, 1.0, 0.0)
        return acc_fl, acc_cnt

    zero = jnp.zeros((_LANES,), jnp.float32)
    acc_fl, acc_cnt = lax.fori_loop(0, _STEPS, body, (zero, zero))
    acc_v[0, :] = acc_fl
    acc_v[1, :] = acc_cnt
    pltpu.sync_copy(acc_v, out_hbm.at[wid])


def _tc_body(x_ref, t_ref, s1_ref, s2_ref):
    i = pl.program_id(0) * pl.num_programs(1) + pl.program_id(1)
    x = x_ref[0, 0, :, :]
    t = t_ref[0, 0, :, :]
    is_one = t == 1
    ns = jnp.where(is_one, -x, x)
    kept = -ns <= _LOGIT_T
    bce = jnp.maximum(x, 0.0) - jnp.where(is_one, x, 0.0) + jnp.log1p(
        jnp.exp(-jnp.abs(x)))
    pt = jnp.exp(-bce)
    om = 1.0 - pt
    fl = (_ALPHA * bce) * (om * om)
    m = kept.astype(jnp.float32)
    ps1 = jnp.sum(fl * m)
    ps2 = jnp.sum(m)

    @pl.when(i == 0)
    def _():
        s1_ref[0, 0] = ps1
        s2_ref[0, 0] = ps2

    @pl.when(i > 0)
    def _():
        s1_ref[0, 0] += ps1
        s2_ref[0, 0] += ps2


def _tc_partial_sums(inp, tgt32, row0, nblocks):
    blk0 = row0 // _TC_BLOCK_ROWS
    idx = lambda b, r: (b, 0, blk0 + r, 0)
    spec = pl.BlockSpec((1, 1, _TC_BLOCK_ROWS, 512), idx)
    return pl.pallas_call(
        _tc_body,
        grid=(4, nblocks),
        in_specs=[spec, spec],
        out_specs=[
            pl.BlockSpec(memory_space=pltpu.SMEM),
            pl.BlockSpec(memory_space=pltpu.SMEM),
        ],
        out_shape=[
            jax.ShapeDtypeStruct((1, 1), jnp.float32),
            jax.ShapeDtypeStruct((1, 1), jnp.float32),
        ],
    )(inp, tgt32)


def _aug_loss(inp, tgt, s1, s2):
    del s1, s2
    x = inp[:, 0, :, :].reshape(-1)
    t = tgt[:, 0, :, :].astype(jnp.float32).reshape(-1)
    probs = jax.nn.sigmoid(x)
    kept = jnp.where(t == 1.0, probs <= _THRESH,
                     jnp.where(t == 0.0, probs >= 1.0 - _THRESH, False))
    hardest = jnp.argsort(jnp.abs(probs - 0.5))[:_MIN_KEPT]
    kept = kept.at[hardest].set(True)
    bce = jnp.maximum(x, 0.0) - x * t + jnp.log1p(jnp.exp(-jnp.abs(x)))
    pt = jnp.exp(-bce)
    fl = _ALPHA * (1.0 - pt) ** 2.0 * bce
    m = kept.astype(jnp.float32)
    return jnp.sum(fl * m) / jnp.sum(m)


def kernel(input, target):
    n_terms = jnp.float32(input.shape[1] - 1)
    tgt32 = target.astype(jnp.int32)
    x_sc = input[:, 0, :_SC_ROWS, :].reshape(_NW, _CHUNK)
    t_sc = tgt32[:, 0, :_SC_ROWS, :].reshape(_NW, _CHUNK)
    parts = _sc_partial_sums(x_sc, t_sc)
    a_s1, a_s2 = _tc_partial_sums(input, tgt32, _TCA_ROW0, _TCA_BLOCKS)
    sc_s1 = jnp.sum(parts[:, 0, :])
    sc_s2 = jnp.sum(parts[:, 1, :])
    b_s1, b_s2 = _tc_partial_sums(input, tgt32, _TCB_ROW0, _TCB_BLOCKS)
    s1 = sc_s1 + a_s1[0, 0] + b_s1[0, 0]
    s2 = sc_s2 + a_s2[0, 0] + b_s2[0, 0]
    loss = lax.cond(s2 < _MIN_KEPT, _aug_loss,
                    lambda inp, tgt, a, b: a / b,
                    input, target, s1, s2)
    return n_terms * loss


# speedup vs baseline: 1.1416x; 1.0697x over previous
import functools

import jax
import jax.numpy as jnp
from jax import lax
from jax.experimental import pallas as pl
from jax.experimental.pallas import tpu as pltpu
from jax.experimental.pallas import tpu_sc as plsc

_THRESH = 0.7
_MIN_KEPT = 10000
_ALPHA = 0.25

_NC = 2
_NS = 16
_NW = _NC * _NS
_LANES = 16
_HW = 512 * 512
_TOTAL = 4 * _HW
_SC_ROWS = 128
_SC_N = 4 * _SC_ROWS * 512
_CHUNK = _SC_N // _NW
_UNROLL = 4
_STEPS = _CHUNK // (_LANES * _UNROLL)
_TC_BLOCK_ROWS = 128
_TC_ROW0 = _SC_ROWS
_TC_BLOCKS = (512 - _TC_ROW0) // _TC_BLOCK_ROWS

_LOGIT_T = 0.8472978603872034


def _focal_and_kept(x, is_one):
    ns = jnp.where(is_one, -x, x)
    s = -ns
    kept = s <= _LOGIT_T
    u = jnp.exp(-jnp.abs(x))
    y = u / (2.0 + u)
    y2 = y * y
    log1pu = y * (2.0 + y2 * (2.0 / 3 + y2 * (2.0 / 5 + y2 * (
        2.0 / 7 + y2 * (2.0 / 9 + y2 * (2.0 / 11))))))
    bce = log1pu + jnp.maximum(ns, 0.0)
    inv = 1.0 / (1.0 + u)
    om = jnp.where(s >= 0.0, u * inv, inv)
    fl = (_ALPHA * bce) * (om * om)
    return fl, kept


@functools.partial(
    pl.kernel,
    out_type=jax.ShapeDtypeStruct((_NW, 2, _LANES), jnp.float32),
    mesh=plsc.VectorSubcoreMesh(core_axis_name="c", subcore_axis_name="s",
                                num_cores=_NC),
    scratch_types=[
        pltpu.VMEM((_CHUNK,), jnp.float32),
        pltpu.VMEM((_CHUNK,), jnp.float32),
        pltpu.VMEM((2, _LANES), jnp.float32),
    ],
)
def _sc_partial_sums(xt_hbm, out_hbm, x_v, t_v, acc_v):
    wid = lax.axis_index("s") * _NC + lax.axis_index("c")
    pltpu.sync_copy(xt_hbm.at[0, wid], x_v)
    pltpu.sync_copy(xt_hbm.at[1, wid], t_v)

    def body(i, carry):
        acc_fl, acc_cnt = carry
        base = i * (_LANES * _UNROLL)
        for k in range(_UNROLL):
            off = base + k * _LANES
            x = x_v[pl.ds(off, _LANES)]
            t = t_v[pl.ds(off, _LANES)]
            fl, kept = _focal_and_kept(x, t == 1.0)
            acc_fl = acc_fl + jnp.where(kept, fl, 0.0)
            acc_cnt = acc_cnt + jnp.where(kept, 1.0, 0.0)
        return acc_fl, acc_cnt

    zero = jnp.zeros((_LANES,), jnp.float32)
    acc_fl, acc_cnt = lax.fori_loop(0, _STEPS, body, (zero, zero))
    acc_v[0, :] = acc_fl
    acc_v[1, :] = acc_cnt
    pltpu.sync_copy(acc_v, out_hbm.at[wid])


def _tc_body(x_ref, t_ref, s1_ref, s2_ref):
    i = pl.program_id(0) * pl.num_programs(1) + pl.program_id(1)
    x = x_ref[0, 0, :, :]
    t = t_ref[0, 0, :, :]
    fl, kept = _focal_and_kept(x, t == 1)
    m = kept.astype(jnp.float32)
    ps1 = jnp.sum(fl * m)
    ps2 = jnp.sum(m)

    @pl.when(i == 0)
    def _():
        s1_ref[0, 0] = ps1
        s2_ref[0, 0] = ps2

    @pl.when(i > 0)
    def _():
        s1_ref[0, 0] += ps1
        s2_ref[0, 0] += ps2


def _tc_partial_sums(inp, tgt32, row0, nblocks):
    blk0 = row0 // _TC_BLOCK_ROWS
    idx = lambda b, r: (b, 0, blk0 + r, 0)
    spec = pl.BlockSpec((1, 1, _TC_BLOCK_ROWS, 512), idx)
    return pl.pallas_call(
        _tc_body,
        grid=(4, nblocks),
        in_specs=[spec, spec],
        out_specs=[
            pl.BlockSpec(memory_space=pltpu.SMEM),
            pl.BlockSpec(memory_space=pltpu.SMEM),
        ],
        out_shape=[
            jax.ShapeDtypeStruct((1, 1), jnp.float32),
            jax.ShapeDtypeStruct((1, 1), jnp.float32),
        ],
    )(inp, tgt32)


def _aug_loss(inp, tgt, s1, s2):
    del s1, s2
    x = inp[:, 0, :, :].reshape(-1)
    t = tgt[:, 0, :, :].astype(jnp.float32).reshape(-1)
    probs = jax.nn.sigmoid(x)
    kept = jnp.where(t == 1.0, probs <= _THRESH,
                     jnp.where(t == 0.0, probs >= 1.0 - _THRESH, False))
    hardest = jnp.argsort(jnp.abs(probs - 0.5))[:_MIN_KEPT]
    kept = kept.at[hardest].set(True)
    bce = jnp.maximum(x, 0.0) - x * t + jnp.log1p(jnp.exp(-jnp.abs(x)))
    pt = jnp.exp(-bce)
    fl = _ALPHA * (1.0 - pt) ** 2.0 * bce
    m = kept.astype(jnp.float32)
    return jnp.sum(fl * m) / jnp.sum(m)


def kernel(input, target):
    n_terms = jnp.float32(input.shape[1] - 1)
    tgt32 = target.astype(jnp.int32)
    x_sc = input[:, 0, :_SC_ROWS, :].reshape(_NW, _CHUNK)
    t_sc = tgt32[:, 0, :_SC_ROWS, :].astype(jnp.float32).reshape(_NW, _CHUNK)
    xt = jnp.stack([x_sc, t_sc])
    parts = _sc_partial_sums(xt)
    tc_s1, tc_s2 = _tc_partial_sums(input, tgt32, _TC_ROW0, _TC_BLOCKS)
    s1 = jnp.sum(parts[:, 0, :]) + tc_s1[0, 0]
    s2 = jnp.sum(parts[:, 1, :]) + tc_s2[0, 0]
    loss = lax.cond(s2 < _MIN_KEPT, _aug_loss,
                    lambda inp, tgt, a, b: a / b,
                    input, target, s1, s2)
    return n_terms * loss


# speedup vs baseline: 1.3810x; 1.2097x over previous
import functools

import jax
import jax.numpy as jnp
from jax import lax
from jax.experimental import pallas as pl
from jax.experimental.pallas import tpu as pltpu
from jax.experimental.pallas import tpu_sc as plsc

_THRESH = 0.7
_MIN_KEPT = 10000
_ALPHA = 0.25

_NC = 2
_NS = 16
_NW = _NC * _NS
_LANES = 16
_HW = 512 * 512
_TOTAL = 4 * _HW
_SC_ROWS = 128
_TC_BLOCK_ROWS = 128
_TC_ROW0 = _SC_ROWS
_TC_BLOCKS = (512 - _TC_ROW0) // _TC_BLOCK_ROWS

_LOGIT_T = 0.8472978603872034


def _focal_and_kept(x, is_one):
    ns = jnp.where(is_one, -x, x)
    s = -ns
    kept = s <= _LOGIT_T
    u = jnp.exp(-jnp.abs(x))
    y = u / (2.0 + u)
    y2 = y * y
    log1pu = y * (2.0 + y2 * (2.0 / 3 + y2 * (2.0 / 5 + y2 * (
        2.0 / 7 + y2 * (2.0 / 9)))))
    bce = log1pu + jnp.maximum(ns, 0.0)
    inv = 1.0 / (1.0 + u)
    om = jnp.where(s >= 0.0, u * inv, inv)
    fl = (_ALPHA * bce) * (om * om)
    return fl, kept


_SC_WROWS = _SC_ROWS // 8


@functools.partial(
    pl.kernel,
    out_type=jax.ShapeDtypeStruct((_NW, 2, _LANES), jnp.float32),
    mesh=plsc.VectorSubcoreMesh(core_axis_name="c", subcore_axis_name="s",
                                num_cores=_NC),
    compiler_params=pltpu.CompilerParams(use_tc_tiling_on_sc=True),
    scratch_types=[
        pltpu.VMEM((_SC_WROWS, 512), jnp.float32),
        pltpu.VMEM((_SC_WROWS, 512), jnp.int32),
        pltpu.VMEM((2, _LANES), jnp.float32),
    ],
)
def _sc_partial_sums(x_hbm, t_hbm, out_hbm, x_v, t_v, acc_v):
    wid = lax.axis_index("s") * _NC + lax.axis_index("c")
    b = wid // 8
    r0 = (wid - b * 8) * _SC_WROWS
    pltpu.sync_copy(x_hbm.at[b, 0, pl.ds(r0, _SC_WROWS), :], x_v)
    pltpu.sync_copy(t_hbm.at[b, 0, pl.ds(r0, _SC_WROWS), :], t_v)

    def body(r, carry):
        acc_fl, acc_cnt = carry
        for k in range(512 // _LANES):
            off = k * _LANES
            x = x_v[r, pl.ds(off, _LANES)]
            t = t_v[r, pl.ds(off, _LANES)]
            fl, kept = _focal_and_kept(x, t == 1)
            acc_fl = acc_fl + jnp.where(kept, fl, 0.0)
            acc_cnt = acc_cnt + jnp.where(kept, 1.0, 0.0)
        return acc_fl, acc_cnt

    zero = jnp.zeros((_LANES,), jnp.float32)
    acc_fl, acc_cnt = lax.fori_loop(0, _SC_WROWS, body, (zero, zero))
    acc_v[0, :] = acc_fl
    acc_v[1, :] = acc_cnt
    pltpu.sync_copy(acc_v, out_hbm.at[wid])


def _tc_body(x_ref, t_ref, s1_ref, s2_ref):
    i = pl.program_id(0) * pl.num_programs(1) + pl.program_id(1)
    x = x_ref[0, 0, :, :]
    t = t_ref[0, 0, :, :]
    fl, kept = _focal_and_kept(x, t == 1)
    m = kept.astype(jnp.float32)
    ps1 = jnp.sum(fl * m)
    ps2 = jnp.sum(m)

    @pl.when(i == 0)
    def _():
        s1_ref[0, 0] = ps1
        s2_ref[0, 0] = ps2

    @pl.when(i > 0)
    def _():
        s1_ref[0, 0] += ps1
        s2_ref[0, 0] += ps2


def _tc_partial_sums(inp, tgt32, row0, nblocks):
    blk0 = row0 // _TC_BLOCK_ROWS
    idx = lambda b, r: (b, 0, blk0 + r, 0)
    spec = pl.BlockSpec((1, 1, _TC_BLOCK_ROWS, 512), idx)
    return pl.pallas_call(
        _tc_body,
        grid=(4, nblocks),
        in_specs=[spec, spec],
        out_specs=[
            pl.BlockSpec(memory_space=pltpu.SMEM),
            pl.BlockSpec(memory_space=pltpu.SMEM),
        ],
        out_shape=[
            jax.ShapeDtypeStruct((1, 1), jnp.float32),
            jax.ShapeDtypeStruct((1, 1), jnp.float32),
        ],
    )(inp, tgt32)


def _aug_loss(inp, tgt, s1, s2):
    del s1, s2
    x = inp[:, 0, :, :].reshape(-1)
    t = tgt[:, 0, :, :].astype(jnp.float32).reshape(-1)
    probs = jax.nn.sigmoid(x)
    kept = jnp.where(t == 1.0, probs <= _THRESH,
                     jnp.where(t == 0.0, probs >= 1.0 - _THRESH, False))
    hardest = jnp.argsort(jnp.abs(probs - 0.5))[:_MIN_KEPT]
    kept = kept.at[hardest].set(True)
    bce = jnp.maximum(x, 0.0) - x * t + jnp.log1p(jnp.exp(-jnp.abs(x)))
    pt = jnp.exp(-bce)
    fl = _ALPHA * (1.0 - pt) ** 2.0 * bce
    m = kept.astype(jnp.float32)
    return jnp.sum(fl * m) / jnp.sum(m)


def kernel(input, target):
    n_terms = jnp.float32(input.shape[1] - 1)
    tgt32 = target.astype(jnp.int32)
    tc_s1, tc_s2 = _tc_partial_sums(input, tgt32, _TC_ROW0, _TC_BLOCKS)
    parts = _sc_partial_sums(input, tgt32)
    ps = jnp.sum(parts, axis=(0, 2))
    s1 = ps[0] + tc_s1[0, 0]
    s2 = ps[1] + tc_s2[0, 0]
    loss = lax.cond(s2 < _MIN_KEPT, _aug_loss,
                    lambda inp, tgt, a, b: a / b,
                    input, target, s1, s2)
    return n_terms * loss
